# baseline (speedup 1.0000x reference)
import functools
import math

import jax
import jax.numpy as jnp
import numpy as np
from jax import lax
from jax.experimental import pallas as pl
from jax.experimental.pallas import tpu as pltpu
from jax.experimental.pallas import tpu_sc as plsc

N_NODES = 50000
N_EDGES = 800000
NUM_BASIS = 16
MAX_RADIUS = 3.15
INV_DEG = 1.0 / math.sqrt(16.0)

_STEP = MAX_RADIUS / (NUM_BASIS + 1)
_HB_SCALE = 1.14136 * math.exp(2.0) * math.sqrt(float(NUM_BASIS))

_CG2 = np.zeros((5, 3, 3), dtype=np.float32)
_s2 = 1.0 / np.sqrt(2.0)
_s6 = 1.0 / np.sqrt(6.0)
_CG2[0, 0, 2] = _CG2[0, 2, 0] = _s2
_CG2[1, 0, 1] = _CG2[1, 1, 0] = _s2
_CG2[2] = np.diag([-1.0, 2.0, -1.0]) * _s6
_CG2[3, 1, 2] = _CG2[3, 2, 1] = _s2
_CG2[4] = np.diag([-1.0, 0.0, 1.0]) * _s2
_CG2F = _CG2.reshape(5, 9)

_EB = 4000
_NB = 2000
_W = 128
_NWIN = N_EDGES // _W
_NSUB = 16
_R8 = 6256
_R4 = 12512


def _silu(x):
    return x * jax.nn.sigmoid(x)


def _dot(a, b):
    return jnp.dot(a, b, preferred_element_type=jnp.float32)


def _slot_pack(vals, slot, nslot):
    w = 128 // nslot
    tiled = jnp.concatenate([vals] * nslot, axis=1)
    lane_slot = lax.broadcasted_iota(jnp.int32, (1, 128), 1) // w
    return tiled * (slot == lane_slot).astype(jnp.float32)


def _geom(ev):
    r = jnp.sqrt(jnp.sum(ev * ev, axis=1, keepdims=True))
    jgrid = lax.broadcasted_iota(
        jnp.int32, (1, NUM_BASIS), 1).astype(jnp.float32) + 1.0
    d = r / _STEP - jgrid
    t0 = d + 1.0
    t1 = 1.0 - d
    e0 = jnp.where(t0 > 0.0, jnp.exp(-1.0 / jnp.where(t0 > 0.0, t0, 1.0)), 0.0)
    e1 = jnp.where(t1 > 0.0, jnp.exp(-1.0 / jnp.where(t1 > 0.0, t1, 1.0)), 0.0)
    hb = _HB_SCALE * e0 * e1
    u = ev / jnp.maximum(r, 1e-9)
    return hb, u


def _radial(hb, A, B):
    h = _silu(_dot(hb, A[...]) * (1.0 / math.sqrt(A.shape[0])))
    return _dot(h, B[...]) * (1.0 / math.sqrt(B.shape[0]))


def _node0_body(x_ref, emb, l10, sc0, xs_tbl_ref, sc1s_ref):
    xb = x_ref[...]
    oh = (xb == lax.broadcasted_iota(jnp.int32, (1, 4), 1)).astype(jnp.float32)
    xs = _dot(oh, _dot(emb[...], l10[...]))
    xs_tbl_ref[...] = jnp.concatenate(
        [xs, jnp.zeros((xs.shape[0], 127), jnp.float32)], axis=1)
    sc1s_ref[...] = _dot(oh, _dot(emb[...], sc0[...]))


def _edgeA_body(ev_ref, xsg_ref, dst_ref, a1, b1, pl1_ref):
    ev = ev_ref[...]
    hb, u = _geom(ev)
    y1 = math.sqrt(3.0) * u
    w1 = _radial(hb, a1, b1)
    nb = ev.shape[0]
    xs = xsg_ref[...][:, 0:1]
    ms = INV_DEG * w1[:, 0:1] * xs
    mv = (INV_DEG * w1[:, 1:2] * xs) * y1
    p16 = jnp.concatenate([ms, mv, jnp.zeros((nb, 12), jnp.float32)], axis=1)
    slot8 = jnp.bitwise_and(dst_ref[...], 7)
    pl1_ref[...] = _slot_pack(p16, slot8, 8)


def _node1_body(acc_ref, sc1s_ref, l20, l21, l10, l11, sc0, sc1,
                tbl2_ref, sc2s_ref, sc2v_ref):
    acc = acc_ref[...]
    ns = acc[:, 0:1]
    s_out = sc1s_ref[...] + _dot(ns, l20[...])
    vox = _dot(acc[:, 1:2], l21[...])
    voy = _dot(acc[:, 2:3], l21[...])
    voz = _dot(acc[:, 3:4], l21[...])
    sh = _silu(s_out[:, :8])
    g = jax.nn.sigmoid(s_out[:, 8:16])
    vhx, vhy, vhz = vox * g, voy * g, voz * g
    r8 = 1.0 / math.sqrt(8.0)
    nb = acc.shape[0]
    tbl2_ref[...] = jnp.concatenate(
        [_dot(sh, l10[...]) * r8, _dot(vhx, l11[...]) * r8,
         _dot(vhy, l11[...]) * r8, _dot(vhz, l11[...]) * r8,
         jnp.zeros((nb, 96), jnp.float32)], axis=1)
    sc2s_ref[...] = _dot(sh, sc0[...]) * r8
    sc2v_ref[...] = jnp.concatenate(
        [_dot(vhx, sc1[...]), _dot(vhy, sc1[...]),
         _dot(vhz, sc1[...])], axis=1) * r8


def _edgeL2_body(ev_ref, g2_ref, dst_ref, cg, a2, b2, l20, l21,
                pa_ref, pb_ref):
    ev = ev_ref[...]
    hb, u = _geom(ev)
    ux, uy, uz = u[:, 0:1], u[:, 1:2], u[:, 2:3]
    s3 = math.sqrt(3.0)
    y1x, y1y, y1z = s3 * ux, s3 * uy, s3 * uz
    s15 = math.sqrt(15.0)
    y2 = jnp.concatenate([
        s15 * ux * uz,
        s15 * ux * uy,
        (math.sqrt(5.0) / 2.0) * (3.0 * uy * uy - 1.0),
        s15 * uy * uz,
        (s15 / 2.0) * (uz * uz - ux * ux),
    ], axis=1)
    me = _dot(y2, cg[...])
    w2 = _radial(hb, a2, b2)
    g2 = g2_ref[...]
    w0, wa = w2[:, 0:8], w2[:, 8:16]
    wb, w3 = w2[:, 16:24], w2[:, 24:32]
    w4, w5 = w2[:, 32:40], w2[:, 40:48]
    s1 = g2[:, 0:8]
    v1x, v1y, v1z = g2[:, 8:16], g2[:, 16:24], g2[:, 24:32]

    p0 = w0 * s1
    dot = v1x * y1x + v1y * y1y + v1z * y1z
    p3 = w3 * dot * (1.0 / math.sqrt(3.0))
    was = wa * s1
    pax, pay, paz = was * y1x, was * y1y, was * y1z
    pbx, pby, pbz = wb * v1x, wb * v1y, wb * v1z
    rs2 = 1.0 / math.sqrt(2.0)
    p4x = w4 * (v1y * y1z - v1z * y1y) * rs2
    p4y = w4 * (v1z * y1x - v1x * y1z) * rs2
    p4z = w4 * (v1x * y1y - v1y * y1x) * rs2
    p5x = w5 * (me[:, 0:1] * v1x + me[:, 1:2] * v1y + me[:, 2:3] * v1z)
    p5y = w5 * (me[:, 3:4] * v1x + me[:, 4:5] * v1y + me[:, 5:6] * v1z)
    p5z = w5 * (me[:, 6:7] * v1x + me[:, 7:8] * v1y + me[:, 8:9] * v1z)

    msc = _dot(jnp.concatenate([p0, p3], axis=1), l20[...]) * 0.25
    r32 = 1.0 / math.sqrt(32.0)
    mvx = _dot(jnp.concatenate([pax, pbx, p4x, p5x], axis=1), l21[...]) * r32
    mvy = _dot(jnp.concatenate([pay, pby, p4y, p5y], axis=1), l21[...]) * r32
    mvz = _dot(jnp.concatenate([paz, pbz, p4z, p5z], axis=1), l21[...]) * r32

    nb = msc.shape[0]
    dstb = dst_ref[...]
    p32 = INV_DEG * jnp.concatenate([msc, mvx, mvy], axis=1)
    p16 = jnp.concatenate(
        [INV_DEG * mvz, jnp.zeros((nb, 8), jnp.float32)], axis=1)
    pa_ref[...] = _slot_pack(p32, jnp.bitwise_and(dstb, 3), 4)
    pb_ref[...] = _slot_pack(p16, jnp.bitwise_and(dstb, 7), 8)


def _node2_body(accA_ref, accB_ref, sc2s_ref, sc2v_ref,
                l10, l11, sc0, tbl3_ref, sc3s_ref):
    accA = accA_ref[...]
    s_out = sc2s_ref[...] + accA[:, 0:16]
    sc2v = sc2v_ref[...]
    vox = sc2v[:, 0:8] + accA[:, 16:24]
    voy = sc2v[:, 8:16] + accA[:, 24:32]
    voz = sc2v[:, 16:24] + accB_ref[...][:, 0:8]
    sh = _silu(s_out[:, :8])
    g = jax.nn.sigmoid(s_out[:, 8:16])
    vhx, vhy, vhz = vox * g, voy * g, voz * g
    r8 = 1.0 / math.sqrt(8.0)
    nb = accA.shape[0]
    tbl3_ref[...] = jnp.concatenate(
        [_dot(sh, l10[...]) * r8, _dot(vhx, l11[...]) * r8,
         _dot(vhy, l11[...]) * r8, _dot(vhz, l11[...]) * r8,
         jnp.zeros((nb, 96), jnp.float32)], axis=1)
    sc3s_ref[...] = _dot(sh, sc0[...]) * r8


def _edgeL3_body(ev_ref, g3_ref, dst_ref, a3, b3, pl3_ref):
    ev = ev_ref[...]
    hb, u = _geom(ev)
    s3 = math.sqrt(3.0)
    y1x, y1y, y1z = s3 * u[:, 0:1], s3 * u[:, 1:2], s3 * u[:, 2:3]
    w3l = _radial(hb, a3, b3)
    g3 = g3_ref[...]
    w3a, w3b = w3l[:, 0:8], w3l[:, 8:16]
    s1 = g3[:, 0:8]
    v1x, v1y, v1z = g3[:, 8:16], g3[:, 16:24], g3[:, 24:32]
    p0 = w3a * s1
    dot = v1x * y1x + v1y * y1y + v1z * y1z
    p3d = w3b * dot * (1.0 / math.sqrt(3.0))
    p16 = INV_DEG * jnp.concatenate([p0, p3d], axis=1)
    pl3_ref[...] = _slot_pack(p16, jnp.bitwise_and(dst_ref[...], 7), 8)


def _node3_body(acc3_ref, sc3s_ref, l20, w1, w2, out_ref):
    ns = acc3_ref[...]
    h = sc3s_ref[...] + _dot(ns, l20[...]) * 0.25
    h1 = _silu(_dot(h, w1[...]) * 0.125)
    out_ref[...] = _dot(h1, w2[...]) * 0.125


def _tc_call(body, grid_n, blk, ins, in_widths, out_widths, weight_args):
    grid = (grid_n,)

    def rb(c):
        return pl.BlockSpec((blk, c), lambda i: (i, 0))

    def full(a):
        return pl.BlockSpec(a.shape, lambda i: (0, 0))

    n_rows = grid_n * blk
    out_shapes = [jax.ShapeDtypeStruct((n_rows, c), jnp.float32)
                  for c in out_widths]
    return pl.pallas_call(
        body,
        grid=grid,
        in_specs=[rb(c) for c in in_widths] + [full(w) for w in weight_args],
        out_specs=[rb(c) for c in out_widths],
        out_shape=out_shapes,
    )(*ins, *weight_args)


_MESH = plsc.VectorSubcoreMesh(core_axis_name="c", subcore_axis_name="s")


def _stripe_sizes(R):
    base = ((R // _NSUB) // 8) * 8 + 8
    return base, R - 15 * base


def _striped(sid, R, fn):
    base, tail = _stripe_sizes(R)

    @pl.when(sid < _NSUB - 1)
    def _main():
        fn(sid * base, base)

    @pl.when(sid == _NSUB - 1)
    def _tail():
        fn(15 * base, tail)


def _sc_gather(table, idx_flat):
    npw = 196

    @functools.partial(
        pl.kernel, mesh=_MESH,
        out_type=jax.ShapeDtypeStruct((N_EDGES, 128), jnp.float32),
        scratch_types=[pltpu.VMEM((_W,), jnp.int32),
                       pltpu.VMEM((_W, 128), jnp.float32),
                       pltpu.SemaphoreType.DMA],
    )
    def gk(table_hbm, idx_hbm, out_hbm, idx_v, rows_v, sem):
        wid = lax.axis_index("c") * _NSUB + lax.axis_index("s")
        start = (wid * (_NWIN - npw)) // 31

        def body(j, carry):
            win = start + j
            pltpu.sync_copy(idx_hbm.at[pl.ds(win * _W, _W)], idx_v)
            pltpu.async_copy(table_hbm.at[idx_v], rows_v, sem).wait()
            pltpu.sync_copy(rows_v, out_hbm.at[pl.ds(win * _W, _W)])
            return carry

        lax.fori_loop(0, npw, body, 0)

    return gk(table, idx_flat)


def _sc_scatter_split(payload, idx_flat, zeros):
    @functools.partial(
        pl.kernel, mesh=_MESH,
        out_type=[jax.ShapeDtypeStruct((_R8, 128), jnp.float32),
                  jax.ShapeDtypeStruct((_R8, 128), jnp.float32)],
        scratch_types=[pltpu.VMEM((_W,), jnp.int32),
                       pltpu.VMEM((_W, 128), jnp.float32),
                       pltpu.VMEM_SHARED((_R8, 128), jnp.float32)],
    )
    def sk(pl_hbm, idx_hbm, zero_hbm, out0, out1, idx_v, vals_v, accum):
        cid = lax.axis_index("c")
        sid = lax.axis_index("s")
        wid = cid * _NSUB + sid
        _striped(sid, _R8, lambda off, sz: pltpu.sync_copy(
            zero_hbm.at[pl.ds(0, sz)], accum.at[pl.ds(off, sz)]))
        plsc.subcore_barrier()

        def do_win(win):
            pltpu.sync_copy(idx_hbm.at[pl.ds(win * _W, _W)], idx_v)
            pltpu.sync_copy(pl_hbm.at[pl.ds(win * _W, _W)], vals_v)
            pltpu.sync_copy(vals_v, accum.at[idx_v], add=True)

        def body(j, carry):
            do_win(wid + 32 * j)
            return carry

        lax.fori_loop(0, _NWIN // 32, body, 0)

        @pl.when(wid < _NWIN % 32)
        def _extra():
            do_win((_NWIN // 32) * 32 + wid)

        plsc.subcore_barrier()

        @pl.when(cid == 0)
        def _w0():
            _striped(sid, _R8, lambda off, sz: pltpu.sync_copy(
                accum.at[pl.ds(off, sz)], out0.at[pl.ds(off, sz)]))

        @pl.when(cid == 1)
        def _w1():
            _striped(sid, _R8, lambda off, sz: pltpu.sync_copy(
                accum.at[pl.ds(off, sz)], out1.at[pl.ds(off, sz)]))

    return sk(payload, idx_flat, zeros)


def _sc_scatter_l2(pa, idx4, pb, idx8, zeros):
    @functools.partial(
        pl.kernel, mesh=_MESH,
        out_type=[jax.ShapeDtypeStruct((_R4, 128), jnp.float32),
                  jax.ShapeDtypeStruct((_R8, 128), jnp.float32)],
        scratch_types=[pltpu.VMEM((_W,), jnp.int32),
                       pltpu.VMEM((_W, 128), jnp.float32),
                       pltpu.VMEM_SHARED((_R4, 128), jnp.float32)],
    )
    def sk(pa_hbm, idx4_hbm, pb_hbm, idx8_hbm, zero_hbm,
           outA, outB, idx_v, vals_v, accum):
        cid = lax.axis_index("c")
        sid = lax.axis_index("s")

        @pl.when(cid == 0)
        def _z0():
            _striped(sid, _R4, lambda off, sz: pltpu.sync_copy(
                zero_hbm.at[pl.ds(0, sz)], accum.at[pl.ds(off, sz)]))

        @pl.when(cid == 1)
        def _z1():
            _striped(sid, _R8, lambda off, sz: pltpu.sync_copy(
                zero_hbm.at[pl.ds(0, sz)], accum.at[pl.ds(off, sz)]))

        plsc.subcore_barrier()

        def do_win(p_hbm, i_hbm, win):
            pltpu.sync_copy(i_hbm.at[pl.ds(win * _W, _W)], idx_v)
            pltpu.sync_copy(p_hbm.at[pl.ds(win * _W, _W)], vals_v)
            pltpu.sync_copy(vals_v, accum.at[idx_v], add=True)

        def run(p_hbm, i_hbm):
            def body(j, carry):
                do_win(p_hbm, i_hbm, sid + _NSUB * j)
                return carry
            lax.fori_loop(0, _NWIN // _NSUB, body, 0)

            @pl.when(sid < _NWIN % _NSUB)
            def _():
                do_win(p_hbm, i_hbm, (_NWIN // _NSUB) * _NSUB + sid)

        @pl.when(cid == 0)
        def _c0():
            run(pa_hbm, idx4_hbm)

        @pl.when(cid == 1)
        def _c1():
            run(pb_hbm, idx8_hbm)

        plsc.subcore_barrier()

        @pl.when(cid == 0)
        def _w0():
            _striped(sid, _R4, lambda off, sz: pltpu.sync_copy(
                accum.at[pl.ds(off, sz)], outA.at[pl.ds(off, sz)]))

        @pl.when(cid == 1)
        def _w1():
            _striped(sid, _R8, lambda off, sz: pltpu.sync_copy(
                accum.at[pl.ds(off, sz)], outB.at[pl.ds(off, sz)]))

    return sk(pa, idx4, pb, idx8, zeros)


def _unpack(acc, width):
    return acc.reshape(-1, width)[:N_NODES]


def kernel(x, edge_index, edge_vec, params):
    p = params
    src = edge_index[0].astype(jnp.int32)
    dst = edge_index[1].astype(jnp.int32)
    idx8 = jnp.right_shift(dst, 3)
    idx4 = jnp.right_shift(dst, 2)
    dst_col = dst.reshape(N_EDGES, 1)
    x2 = x.astype(jnp.int32).reshape(N_NODES, 1)
    zeros = jnp.zeros((_stripe_sizes(_R4)[0], 128), jnp.float32)
    cg = jnp.asarray(_CG2F)

    ngrid = N_NODES // _NB
    egrid = N_EDGES // _EB

    xs_tbl, sc1s = _tc_call(
        _node0_body, ngrid, _NB, [x2], [1], [128, 16],
        [p['emb'], p['c1_l10'], p['c1_sc0']])

    xs_g = _sc_gather(xs_tbl, src)
    (pl1,) = _tc_call(
        _edgeA_body, egrid, _EB, [edge_vec, xs_g, dst_col], [3, 128, 1],
        [128],
        [p['c1_fcA'], p['c1_fcB']])
    acc1a, acc1b = _sc_scatter_split(pl1, idx8, zeros)
    acc1 = _unpack(acc1a + acc1b, 16)

    tbl2, sc2s, sc2v = _tc_call(
        _node1_body, ngrid, _NB, [acc1, sc1s], [16, 16], [128, 16, 24],
        [p['c1_l20'], p['c1_l21'], p['c2_l10'], p['c2_l11'],
         p['c2_sc0'], p['c2_sc1']])

    g2 = _sc_gather(tbl2, src)
    pa, pb = _tc_call(
        _edgeL2_body, egrid, _EB, [edge_vec, g2, dst_col], [3, 128, 1],
        [128, 128],
        [cg, p['c2_fcA'], p['c2_fcB'], p['c2_l20'], p['c2_l21']])
    accA, accB = _sc_scatter_l2(pa, idx4, pb, idx8, zeros)
    accA = _unpack(accA, 32)
    accB = _unpack(accB, 16)

    tbl3, sc3s = _tc_call(
        _node2_body, ngrid, _NB, [accA, accB, sc2s, sc2v],
        [32, 16, 16, 24], [128, 64],
        [p['c3_l10'], p['c3_l11'], p['c3_sc0']])

    g3 = _sc_gather(tbl3, src)
    (pl3,) = _tc_call(
        _edgeL3_body, egrid, _EB, [edge_vec, g3, dst_col], [3, 128, 1], [128],
        [p['c3_fcA'], p['c3_fcB']])
    acc3a, acc3b = _sc_scatter_split(pl3, idx8, zeros)
    acc3 = _unpack(acc3a + acc3b, 16)

    (out,) = _tc_call(
        _node3_body, ngrid, _NB, [acc3, sc3s], [16, 64], [4],
        [p['c3_l20'], p['head_W1'], p['head_W2']])
    return out


# speedup vs baseline: 1.0431x; 1.0431x over previous
import functools
import math

import jax
import jax.numpy as jnp
import numpy as np
from jax import lax
from jax.experimental import pallas as pl
from jax.experimental.pallas import tpu as pltpu
from jax.experimental.pallas import tpu_sc as plsc

N_NODES = 50000
N_EDGES = 800000
NUM_BASIS = 16
MAX_RADIUS = 3.15
INV_DEG = 1.0 / math.sqrt(16.0)

_STEP = MAX_RADIUS / (NUM_BASIS + 1)
_HB_SCALE = 1.14136 * math.exp(2.0) * math.sqrt(float(NUM_BASIS))

_CG2 = np.zeros((5, 3, 3), dtype=np.float32)
_s2 = 1.0 / np.sqrt(2.0)
_s6 = 1.0 / np.sqrt(6.0)
_CG2[0, 0, 2] = _CG2[0, 2, 0] = _s2
_CG2[1, 0, 1] = _CG2[1, 1, 0] = _s2
_CG2[2] = np.diag([-1.0, 2.0, -1.0]) * _s6
_CG2[3, 1, 2] = _CG2[3, 2, 1] = _s2
_CG2[4] = np.diag([-1.0, 0.0, 1.0]) * _s2
_CG2F = _CG2.reshape(5, 9)

_EB = 4000
_NB = 2000
_W = 128
_NWIN = N_EDGES // _W
_NSUB = 16
_R8 = 6256
_R4 = 12512


def _silu(x):
    return x * jax.nn.sigmoid(x)


def _dot(a, b):
    return jnp.dot(a, b, preferred_element_type=jnp.float32)


def _slot_pack(vals, slot, nslot):
    w = 128 // nslot
    tiled = jnp.concatenate([vals] * nslot, axis=1)
    lane_slot = lax.broadcasted_iota(jnp.int32, (1, 128), 1) // w
    return tiled * (slot == lane_slot).astype(jnp.float32)


def _node0_body(x_ref, emb, l10, sc0, xs_tbl_ref, sc1s_ref):
    xb = x_ref[...]
    oh = (xb == lax.broadcasted_iota(jnp.int32, (1, 4), 1)).astype(jnp.float32)
    xs = _dot(oh, _dot(emb[...], l10[...]))
    xs_tbl_ref[...] = jnp.concatenate(
        [xs, jnp.zeros((xs.shape[0], 127), jnp.float32)], axis=1)
    sc1s_ref[...] = _dot(oh, _dot(emb[...], sc0[...]))


def _edgeA_body(ev_ref, xsg_ref, dst_ref, cg, a1, b1, a2, b2, a3, b3,
                pl1_ref, pk2_ref, pk3_ref):
    ev = ev_ref[...]
    r = jnp.sqrt(jnp.sum(ev * ev, axis=1, keepdims=True))
    jgrid = lax.broadcasted_iota(
        jnp.int32, (1, NUM_BASIS), 1).astype(jnp.float32) + 1.0
    d = r / _STEP - jgrid
    t0 = d + 1.0
    t1 = 1.0 - d
    e0 = jnp.where(t0 > 0.0, jnp.exp(-1.0 / jnp.where(t0 > 0.0, t0, 1.0)), 0.0)
    e1 = jnp.where(t1 > 0.0, jnp.exp(-1.0 / jnp.where(t1 > 0.0, t1, 1.0)), 0.0)
    hb = _HB_SCALE * e0 * e1

    u = ev / jnp.maximum(r, 1e-9)
    ux, uy, uz = u[:, 0:1], u[:, 1:2], u[:, 2:3]
    y1 = math.sqrt(3.0) * u
    s15 = math.sqrt(15.0)
    y2 = jnp.concatenate([
        s15 * ux * uz,
        s15 * ux * uy,
        (math.sqrt(5.0) / 2.0) * (3.0 * uy * uy - 1.0),
        s15 * uy * uz,
        (s15 / 2.0) * (uz * uz - ux * ux),
    ], axis=1)
    me = _dot(y2, cg[...])

    def radial(A, B):
        h = _silu(_dot(hb, A[...]) * (1.0 / math.sqrt(A.shape[0])))
        return _dot(h, B[...]) * (1.0 / math.sqrt(B.shape[0]))

    w1 = radial(a1, b1)
    w2 = radial(a2, b2)
    w3 = radial(a3, b3)

    nb = ev.shape[0]
    xs = xsg_ref[...][:, 0:1]
    ms = INV_DEG * w1[:, 0:1] * xs
    mv = (INV_DEG * w1[:, 1:2] * xs) * y1
    p16 = jnp.concatenate([ms, mv, jnp.zeros((nb, 12), jnp.float32)], axis=1)
    slot8 = jnp.bitwise_and(dst_ref[...], 7)
    pl1_ref[...] = _slot_pack(p16, slot8, 8)
    pk2_ref[...] = jnp.concatenate(
        [w2, y1, me, jnp.zeros((nb, 4), jnp.float32)], axis=1)
    pk3_ref[...] = jnp.concatenate(
        [w3, y1, jnp.zeros((nb, 5), jnp.float32)], axis=1)


def _node1_body(acc_ref, sc1s_ref, l20, l21, l10, l11, sc0, sc1,
                tbl2_ref, sc2s_ref, sc2v_ref):
    acc = acc_ref[...]
    ns = acc[:, 0:1]
    s_out = sc1s_ref[...] + _dot(ns, l20[...])
    vox = _dot(acc[:, 1:2], l21[...])
    voy = _dot(acc[:, 2:3], l21[...])
    voz = _dot(acc[:, 3:4], l21[...])
    sh = _silu(s_out[:, :8])
    g = jax.nn.sigmoid(s_out[:, 8:16])
    vhx, vhy, vhz = vox * g, voy * g, voz * g
    r8 = 1.0 / math.sqrt(8.0)
    nb = acc.shape[0]
    tbl2_ref[...] = jnp.concatenate(
        [_dot(sh, l10[...]) * r8, _dot(vhx, l11[...]) * r8,
         _dot(vhy, l11[...]) * r8, _dot(vhz, l11[...]) * r8,
         jnp.zeros((nb, 96), jnp.float32)], axis=1)
    sc2s_ref[...] = _dot(sh, sc0[...]) * r8
    sc2v_ref[...] = jnp.concatenate(
        [_dot(vhx, sc1[...]), _dot(vhy, sc1[...]),
         _dot(vhz, sc1[...])], axis=1) * r8


def _edgeL2_body(pk2_ref, g2_ref, dst_ref, l20, l21, pa_ref, pb_ref):
    pk = pk2_ref[...]
    g2 = g2_ref[...]
    w0, wa = pk[:, 0:8], pk[:, 8:16]
    wb, w3 = pk[:, 16:24], pk[:, 24:32]
    w4, w5 = pk[:, 32:40], pk[:, 40:48]
    y1x, y1y, y1z = pk[:, 48:49], pk[:, 49:50], pk[:, 50:51]
    me = pk[:, 51:60]
    s1 = g2[:, 0:8]
    v1x, v1y, v1z = g2[:, 8:16], g2[:, 16:24], g2[:, 24:32]

    p0 = w0 * s1
    dot = v1x * y1x + v1y * y1y + v1z * y1z
    p3 = w3 * dot * (1.0 / math.sqrt(3.0))
    was = wa * s1
    pax, pay, paz = was * y1x, was * y1y, was * y1z
    pbx, pby, pbz = wb * v1x, wb * v1y, wb * v1z
    rs2 = 1.0 / math.sqrt(2.0)
    p4x = w4 * (v1y * y1z - v1z * y1y) * rs2
    p4y = w4 * (v1z * y1x - v1x * y1z) * rs2
    p4z = w4 * (v1x * y1y - v1y * y1x) * rs2
    p5x = w5 * (me[:, 0:1] * v1x + me[:, 1:2] * v1y + me[:, 2:3] * v1z)
    p5y = w5 * (me[:, 3:4] * v1x + me[:, 4:5] * v1y + me[:, 5:6] * v1z)
    p5z = w5 * (me[:, 6:7] * v1x + me[:, 7:8] * v1y + me[:, 8:9] * v1z)

    msc = _dot(jnp.concatenate([p0, p3], axis=1), l20[...]) * 0.25
    r32 = 1.0 / math.sqrt(32.0)
    mvx = _dot(jnp.concatenate([pax, pbx, p4x, p5x], axis=1), l21[...]) * r32
    mvy = _dot(jnp.concatenate([pay, pby, p4y, p5y], axis=1), l21[...]) * r32
    mvz = _dot(jnp.concatenate([paz, pbz, p4z, p5z], axis=1), l21[...]) * r32

    nb = msc.shape[0]
    dstb = dst_ref[...]
    p32 = INV_DEG * jnp.concatenate([msc, mvx, mvy], axis=1)
    p16 = jnp.concatenate(
        [INV_DEG * mvz, jnp.zeros((nb, 8), jnp.float32)], axis=1)
    pa_ref[...] = _slot_pack(p32, jnp.bitwise_and(dstb, 3), 4)
    pb_ref[...] = _slot_pack(p16, jnp.bitwise_and(dstb, 7), 8)


def _node2_body(accA_ref, accB_ref, sc2s_ref, sc2v_ref,
                l10, l11, sc0, tbl3_ref, sc3s_ref):
    accA = accA_ref[...]
    s_out = sc2s_ref[...] + accA[:, 0:16]
    sc2v = sc2v_ref[...]
    vox = sc2v[:, 0:8] + accA[:, 16:24]
    voy = sc2v[:, 8:16] + accA[:, 24:32]
    voz = sc2v[:, 16:24] + accB_ref[...][:, 0:8]
    sh = _silu(s_out[:, :8])
    g = jax.nn.sigmoid(s_out[:, 8:16])
    vhx, vhy, vhz = vox * g, voy * g, voz * g
    r8 = 1.0 / math.sqrt(8.0)
    nb = accA.shape[0]
    tbl3_ref[...] = jnp.concatenate(
        [_dot(sh, l10[...]) * r8, _dot(vhx, l11[...]) * r8,
         _dot(vhy, l11[...]) * r8, _dot(vhz, l11[...]) * r8,
         jnp.zeros((nb, 96), jnp.float32)], axis=1)
    sc3s_ref[...] = _dot(sh, sc0[...]) * r8


def _edgeL3_body(pk3_ref, g3_ref, dst_ref, pl3_ref):
    pk = pk3_ref[...]
    g3 = g3_ref[...]
    w3a, w3b = pk[:, 0:8], pk[:, 8:16]
    y1x, y1y, y1z = pk[:, 16:17], pk[:, 17:18], pk[:, 18:19]
    s1 = g3[:, 0:8]
    v1x, v1y, v1z = g3[:, 8:16], g3[:, 16:24], g3[:, 24:32]
    p0 = w3a * s1
    dot = v1x * y1x + v1y * y1y + v1z * y1z
    p3d = w3b * dot * (1.0 / math.sqrt(3.0))
    p16 = INV_DEG * jnp.concatenate([p0, p3d], axis=1)
    pl3_ref[...] = _slot_pack(p16, jnp.bitwise_and(dst_ref[...], 7), 8)


def _node3_body(acc3_ref, sc3s_ref, l20, w1, w2, out_ref):
    ns = acc3_ref[...]
    h = sc3s_ref[...] + _dot(ns, l20[...]) * 0.25
    h1 = _silu(_dot(h, w1[...]) * 0.125)
    out_ref[...] = _dot(h1, w2[...]) * 0.125


def _tc_call(body, grid_n, blk, ins, in_widths, out_widths, weight_args):
    grid = (grid_n,)

    def rb(c):
        return pl.BlockSpec((blk, c), lambda i: (i, 0))

    def full(a):
        return pl.BlockSpec(a.shape, lambda i: (0, 0))

    n_rows = grid_n * blk
    out_shapes = [jax.ShapeDtypeStruct((n_rows, c), jnp.float32)
                  for c in out_widths]
    return pl.pallas_call(
        body,
        grid=grid,
        in_specs=[rb(c) for c in in_widths] + [full(w) for w in weight_args],
        out_specs=[rb(c) for c in out_widths],
        out_shape=out_shapes,
    )(*ins, *weight_args)


_MESH = plsc.VectorSubcoreMesh(core_axis_name="c", subcore_axis_name="s")


def _stripe_sizes(R):
    base = ((R // _NSUB) // 8) * 8 + 8
    return base, R - 15 * base


def _striped(sid, R, fn):
    base, tail = _stripe_sizes(R)

    @pl.when(sid < _NSUB - 1)
    def _main():
        fn(sid * base, base)

    @pl.when(sid == _NSUB - 1)
    def _tail():
        fn(15 * base, tail)


def _sc_gather(table, idx_flat):
    npw = 196

    @functools.partial(
        pl.kernel, mesh=_MESH,
        out_type=jax.ShapeDtypeStruct((N_EDGES, 128), jnp.float32),
        scratch_types=[pltpu.VMEM((_W,), jnp.int32),
                       pltpu.VMEM((_W, 128), jnp.float32),
                       pltpu.SemaphoreType.DMA],
    )
    def gk(table_hbm, idx_hbm, out_hbm, idx_v, rows_v, sem):
        wid = lax.axis_index("c") * _NSUB + lax.axis_index("s")
        start = (wid * (_NWIN - npw)) // 31

        def body(j, carry):
            win = start + j
            pltpu.sync_copy(idx_hbm.at[pl.ds(win * _W, _W)], idx_v)
            pltpu.async_copy(table_hbm.at[idx_v], rows_v, sem).wait()
            pltpu.sync_copy(rows_v, out_hbm.at[pl.ds(win * _W, _W)])
            return carry

        lax.fori_loop(0, npw, body, 0)

    return gk(table, idx_flat)


def _sc_scatter_split(payload, idx_flat, zeros):
    @functools.partial(
        pl.kernel, mesh=_MESH,
        out_type=[jax.ShapeDtypeStruct((_R8, 128), jnp.float32),
                  jax.ShapeDtypeStruct((_R8, 128), jnp.float32)],
        scratch_types=[pltpu.VMEM((_W,), jnp.int32),
                       pltpu.VMEM((_W, 128), jnp.float32),
                       pltpu.VMEM_SHARED((_R8, 128), jnp.float32)],
    )
    def sk(pl_hbm, idx_hbm, zero_hbm, out0, out1, idx_v, vals_v, accum):
        cid = lax.axis_index("c")
        sid = lax.axis_index("s")
        wid = cid * _NSUB + sid
        _striped(sid, _R8, lambda off, sz: pltpu.sync_copy(
            zero_hbm.at[pl.ds(0, sz)], accum.at[pl.ds(off, sz)]))
        plsc.subcore_barrier()

        def do_win(win):
            pltpu.sync_copy(idx_hbm.at[pl.ds(win * _W, _W)], idx_v)
            pltpu.sync_copy(pl_hbm.at[pl.ds(win * _W, _W)], vals_v)
            pltpu.sync_copy(vals_v, accum.at[idx_v], add=True)

        def body(j, carry):
            do_win(wid + 32 * j)
            return carry

        lax.fori_loop(0, _NWIN // 32, body, 0)

        @pl.when(wid < _NWIN % 32)
        def _extra():
            do_win((_NWIN // 32) * 32 + wid)

        plsc.subcore_barrier()

        @pl.when(cid == 0)
        def _w0():
            _striped(sid, _R8, lambda off, sz: pltpu.sync_copy(
                accum.at[pl.ds(off, sz)], out0.at[pl.ds(off, sz)]))

        @pl.when(cid == 1)
        def _w1():
            _striped(sid, _R8, lambda off, sz: pltpu.sync_copy(
                accum.at[pl.ds(off, sz)], out1.at[pl.ds(off, sz)]))

    return sk(payload, idx_flat, zeros)


def _sc_scatter_l2(pa, idx4, pb, idx8, zeros):
    @functools.partial(
        pl.kernel, mesh=_MESH,
        out_type=[jax.ShapeDtypeStruct((_R4, 128), jnp.float32),
                  jax.ShapeDtypeStruct((_R8, 128), jnp.float32)],
        scratch_types=[pltpu.VMEM((_W,), jnp.int32),
                       pltpu.VMEM((_W, 128), jnp.float32),
                       pltpu.VMEM_SHARED((_R4, 128), jnp.float32)],
    )
    def sk(pa_hbm, idx4_hbm, pb_hbm, idx8_hbm, zero_hbm,
           outA, outB, idx_v, vals_v, accum):
        cid = lax.axis_index("c")
        sid = lax.axis_index("s")

        @pl.when(cid == 0)
        def _z0():
            _striped(sid, _R4, lambda off, sz: pltpu.sync_copy(
                zero_hbm.at[pl.ds(0, sz)], accum.at[pl.ds(off, sz)]))

        @pl.when(cid == 1)
        def _z1():
            _striped(sid, _R8, lambda off, sz: pltpu.sync_copy(
                zero_hbm.at[pl.ds(0, sz)], accum.at[pl.ds(off, sz)]))

        plsc.subcore_barrier()

        def do_win(p_hbm, i_hbm, win):
            pltpu.sync_copy(i_hbm.at[pl.ds(win * _W, _W)], idx_v)
            pltpu.sync_copy(p_hbm.at[pl.ds(win * _W, _W)], vals_v)
            pltpu.sync_copy(vals_v, accum.at[idx_v], add=True)

        def run(p_hbm, i_hbm):
            def body(j, carry):
                do_win(p_hbm, i_hbm, sid + _NSUB * j)
                return carry
            lax.fori_loop(0, _NWIN // _NSUB, body, 0)

            @pl.when(sid < _NWIN % _NSUB)
            def _():
                do_win(p_hbm, i_hbm, (_NWIN // _NSUB) * _NSUB + sid)

        @pl.when(cid == 0)
        def _c0():
            run(pa_hbm, idx4_hbm)

        @pl.when(cid == 1)
        def _c1():
            run(pb_hbm, idx8_hbm)

        plsc.subcore_barrier()

        @pl.when(cid == 0)
        def _w0():
            _striped(sid, _R4, lambda off, sz: pltpu.sync_copy(
                accum.at[pl.ds(off, sz)], outA.at[pl.ds(off, sz)]))

        @pl.when(cid == 1)
        def _w1():
            _striped(sid, _R8, lambda off, sz: pltpu.sync_copy(
                accum.at[pl.ds(off, sz)], outB.at[pl.ds(off, sz)]))

    return sk(pa, idx4, pb, idx8, zeros)


def _unpack(acc, width):
    return acc.reshape(-1, width)[:N_NODES]


def kernel(x, edge_index, edge_vec, params):
    p = params
    src = edge_index[0].astype(jnp.int32)
    dst = edge_index[1].astype(jnp.int32)
    idx8 = jnp.right_shift(dst, 3)
    idx4 = jnp.right_shift(dst, 2)
    dst_col = dst.reshape(N_EDGES, 1)
    x2 = x.astype(jnp.int32).reshape(N_NODES, 1)
    zeros = jnp.zeros((_stripe_sizes(_R4)[0], 128), jnp.float32)
    cg = jnp.asarray(_CG2F)

    ngrid = N_NODES // _NB
    egrid = N_EDGES // _EB

    xs_tbl, sc1s = _tc_call(
        _node0_body, ngrid, _NB, [x2], [1], [128, 16],
        [p['emb'], p['c1_l10'], p['c1_sc0']])

    xs_g = _sc_gather(xs_tbl, src)
    pl1, pk2, pk3 = _tc_call(
        _edgeA_body, egrid, _EB, [edge_vec, xs_g, dst_col], [3, 128, 1],
        [128, 64, 24],
        [cg, p['c1_fcA'], p['c1_fcB'], p['c2_fcA'], p['c2_fcB'],
         p['c3_fcA'], p['c3_fcB']])
    acc1a, acc1b = _sc_scatter_split(pl1, idx8, zeros)
    acc1 = _unpack(acc1a + acc1b, 16)

    tbl2, sc2s, sc2v = _tc_call(
        _node1_body, ngrid, _NB, [acc1, sc1s], [16, 16], [128, 16, 24],
        [p['c1_l20'], p['c1_l21'], p['c2_l10'], p['c2_l11'],
         p['c2_sc0'], p['c2_sc1']])

    g2 = _sc_gather(tbl2, src)
    pa, pb = _tc_call(
        _edgeL2_body, egrid, _EB, [pk2, g2, dst_col], [64, 128, 1],
        [128, 128],
        [p['c2_l20'], p['c2_l21']])
    accA, accB = _sc_scatter_l2(pa, idx4, pb, idx8, zeros)
    accA = _unpack(accA, 32)
    accB = _unpack(accB, 16)

    tbl3, sc3s = _tc_call(
        _node2_body, ngrid, _NB, [accA, accB, sc2s, sc2v],
        [32, 16, 16, 24], [128, 64],
        [p['c3_l10'], p['c3_l11'], p['c3_sc0']])

    g3 = _sc_gather(tbl3, src)
    (pl3,) = _tc_call(
        _edgeL3_body, egrid, _EB, [pk3, g3, dst_col], [24, 128, 1], [128],
        [])
    acc3a, acc3b = _sc_scatter_split(pl3, idx8, zeros)
    acc3 = _unpack(acc3a + acc3b, 16)

    (out,) = _tc_call(
        _node3_body, ngrid, _NB, [acc3, sc3s], [16, 64], [4],
        [p['c3_l20'], p['head_W1'], p['head_W2']])
    return out
